# baseline (speedup 1.0000x reference)
import functools

import jax
import jax.numpy as jnp
from jax.experimental import pallas as pl
from jax.experimental.pallas import tpu as pltpu

_TILE = 2048


def _prep_kernel(cb_ref, c2_ref, cbn_ref, cbg_ref):
    for q in range(cb_ref.shape[0]):
        cb = cb_ref[q]
        c2_ref[q] = jnp.sum(cb * cb, axis=-1)[None, :]
        cbn_ref[q] = cb * -2.0
        cb_hi = cb.astype(jnp.bfloat16)
        cb_lo = (cb - cb_hi.astype(jnp.float32)).astype(jnp.bfloat16)
        cbg_ref[q] = jnp.concatenate([cb_hi, cb_lo], axis=-1)


def _rvq_kernel(x_ref, c2_ref, cbn_ref, cbg_ref, out_ref, loss_ref,
                *, n_steps, inv_count):
    nq, k, d = cbn_ref.shape
    x = x_ref[0]
    residual = x
    acc = jnp.zeros_like(x)
    loss = jnp.float32(0.0)
    rr = jnp.sum(residual * residual, axis=-1, keepdims=True)
    for q in range(nq):
        c2 = c2_ref[q]
        s = jax.lax.dot_general(
            residual, cbn_ref[q], (((1,), (1,)), ((), ())),
            preferred_element_type=jnp.float32,
        )
        dists = (rr + s) + c2
        m = jnp.min(dists, axis=-1, keepdims=True)
        onehot = (dists == m).astype(jnp.bfloat16)
        qhl = jax.lax.dot_general(
            onehot, cbg_ref[q], (((1,), (0,)), ((), ())),
            preferred_element_type=jnp.float32,
        )
        quant = qhl[:, :d] + qhl[:, d:]
        loss = loss + jnp.sum(m)
        acc = acc + quant
        residual = residual - quant
        rr = jnp.sum(residual * residual, axis=-1, keepdims=True)
    out_ref[0] = acc
    i = pl.program_id(0)
    lv = jnp.reshape(loss, (1, 1))

    @pl.when(i == 0)
    def _init():
        loss_ref[...] = lv

    @pl.when(i > 0)
    def _accum():
        loss_ref[...] = loss_ref[...] + lv

    @pl.when(i == n_steps - 1)
    def _finish():
        loss_ref[...] = loss_ref[...] * inv_count


def kernel(faces, face_edges, codebooks):
    del face_edges
    b, n, d = faces.shape
    nq, k, _ = codebooks.shape
    m = b * n
    n_steps = m // _TILE
    blocks_per_batch = n // _TILE
    c2, cbn, cbg = pl.pallas_call(
        _prep_kernel,
        out_shape=[
            jax.ShapeDtypeStruct((nq, 1, k), jnp.float32),
            jax.ShapeDtypeStruct((nq, k, d), jnp.float32),
            jax.ShapeDtypeStruct((nq, k, 2 * d), jnp.bfloat16),
        ],
    )(codebooks)
    out, loss = pl.pallas_call(
        functools.partial(
            _rvq_kernel, n_steps=n_steps, inv_count=1.0 / (m * d)
        ),
        grid=(n_steps,),
        in_specs=[
            pl.BlockSpec(
                (1, _TILE, d),
                lambda i: (i // blocks_per_batch, i % blocks_per_batch, 0),
            ),
            pl.BlockSpec((nq, 1, k), lambda i: (0, 0, 0)),
            pl.BlockSpec((nq, k, d), lambda i: (0, 0, 0)),
            pl.BlockSpec((nq, k, 2 * d), lambda i: (0, 0, 0)),
        ],
        out_specs=[
            pl.BlockSpec(
                (1, _TILE, d),
                lambda i: (i // blocks_per_batch, i % blocks_per_batch, 0),
            ),
            pl.BlockSpec((1, 1), lambda i: (0, 0)),
        ],
        out_shape=[
            jax.ShapeDtypeStruct((b, n, d), jnp.float32),
            jax.ShapeDtypeStruct((1, 1), jnp.float32),
        ],
        compiler_params=pltpu.CompilerParams(
            dimension_semantics=("arbitrary",),
        ),
    )(faces, c2, cbn, cbg)
    return out, loss[0, 0]


# speedup vs baseline: 1.1672x; 1.1672x over previous
import functools

import jax
import jax.numpy as jnp
from jax.experimental import pallas as pl
from jax.experimental.pallas import tpu as pltpu

_TILE = 4096


def _prep_kernel(cb_ref, c2_ref, cbn_ref, cbg_ref):
    for q in range(cb_ref.shape[0]):
        cb = cb_ref[q]
        c2_ref[q] = jnp.sum(cb * cb, axis=-1)[None, :]
        cbn_ref[q] = cb * -2.0
        cb_hi = cb.astype(jnp.bfloat16)
        cb_lo = (cb - cb_hi.astype(jnp.float32)).astype(jnp.bfloat16)
        cbg_ref[q] = jnp.concatenate([cb_hi, cb_lo], axis=-1)


def _rvq_kernel(x_ref, c2_ref, cbn_ref, cbg_ref, out_ref, loss_ref,
                *, n_steps, inv_count):
    nq, k, d = cbn_ref.shape
    x = x_ref[...]
    residual = x
    acc = jnp.zeros_like(x)
    loss = jnp.float32(0.0)
    rr = jnp.sum(residual * residual, axis=-1, keepdims=True)
    for q in range(nq):
        c2 = c2_ref[q]
        s = jax.lax.dot_general(
            residual, cbn_ref[q], (((1,), (1,)), ((), ())),
            preferred_element_type=jnp.float32,
        )
        dists = (rr + s) + c2
        m = jnp.min(dists, axis=-1, keepdims=True)
        onehot = (dists == m).astype(jnp.bfloat16)
        qhl = jax.lax.dot_general(
            onehot, cbg_ref[q], (((1,), (0,)), ((), ())),
            preferred_element_type=jnp.float32,
        )
        quant = qhl[:, :d] + qhl[:, d:]
        loss = loss + jnp.sum(m)
        acc = acc + quant
        residual = residual - quant
        rr = jnp.sum(residual * residual, axis=-1, keepdims=True)
    out_ref[...] = acc
    i = pl.program_id(0)
    lv = jnp.reshape(loss, (1, 1))

    @pl.when(i == 0)
    def _init():
        loss_ref[...] = lv

    @pl.when(i > 0)
    def _accum():
        loss_ref[...] = loss_ref[...] + lv

    @pl.when(i == n_steps - 1)
    def _finish():
        loss_ref[...] = loss_ref[...] * inv_count


def kernel(faces, face_edges, codebooks):
    del face_edges
    b, n, d = faces.shape
    nq, k, _ = codebooks.shape
    m = b * n
    flat = faces.reshape(m, d)
    n_steps = m // _TILE
    c2, cbn, cbg = pl.pallas_call(
        _prep_kernel,
        out_shape=[
            jax.ShapeDtypeStruct((nq, 1, k), jnp.float32),
            jax.ShapeDtypeStruct((nq, k, d), jnp.float32),
            jax.ShapeDtypeStruct((nq, k, 2 * d), jnp.bfloat16),
        ],
    )(codebooks)
    out, loss = pl.pallas_call(
        functools.partial(
            _rvq_kernel, n_steps=n_steps, inv_count=1.0 / (m * d)
        ),
        grid=(n_steps,),
        in_specs=[
            pl.BlockSpec((_TILE, d), lambda i: (i, 0)),
            pl.BlockSpec((nq, 1, k), lambda i: (0, 0, 0)),
            pl.BlockSpec((nq, k, d), lambda i: (0, 0, 0)),
            pl.BlockSpec((nq, k, 2 * d), lambda i: (0, 0, 0)),
        ],
        out_specs=[
            pl.BlockSpec((_TILE, d), lambda i: (i, 0)),
            pl.BlockSpec((1, 1), lambda i: (0, 0)),
        ],
        out_shape=[
            jax.ShapeDtypeStruct((m, d), jnp.float32),
            jax.ShapeDtypeStruct((1, 1), jnp.float32),
        ],
        compiler_params=pltpu.CompilerParams(
            dimension_semantics=("arbitrary",),
        ),
    )(flat, c2, cbn, cbg)
    return out.reshape(b, n, d), loss[0, 0]


# speedup vs baseline: 1.2090x; 1.0358x over previous
import functools

import jax
import jax.numpy as jnp
from jax.experimental import pallas as pl
from jax.experimental.pallas import tpu as pltpu

_TILE = 8192


def _prep_kernel(cb_ref, c2_ref, cbn_ref, cbg_ref):
    for q in range(cb_ref.shape[0]):
        cb = cb_ref[q]
        c2_ref[q] = jnp.sum(cb * cb, axis=-1)[None, :]
        cbn_ref[q] = cb * -2.0
        cb_hi = cb.astype(jnp.bfloat16)
        cb_lo = (cb - cb_hi.astype(jnp.float32)).astype(jnp.bfloat16)
        cbg_ref[q] = jnp.concatenate([cb_hi, cb_lo], axis=-1)


def _rvq_kernel(x_ref, c2_ref, cbn_ref, cbg_ref, out_ref, loss_ref,
                *, n_steps, inv_count):
    nq, k, d = cbn_ref.shape
    x = x_ref[...]
    residual = x
    acc = jnp.zeros_like(x)
    loss = jnp.float32(0.0)
    rr = jnp.sum(residual * residual, axis=-1, keepdims=True)
    for q in range(nq):
        c2 = c2_ref[q]
        s = jax.lax.dot_general(
            residual, cbn_ref[q], (((1,), (1,)), ((), ())),
            preferred_element_type=jnp.float32,
        )
        dists = (rr + s) + c2
        m = jnp.min(dists, axis=-1, keepdims=True)
        onehot = (dists == m).astype(jnp.bfloat16)
        qhl = jax.lax.dot_general(
            onehot, cbg_ref[q], (((1,), (0,)), ((), ())),
            preferred_element_type=jnp.float32,
        )
        quant = qhl[:, :d] + qhl[:, d:]
        loss = loss + jnp.sum(m)
        acc = acc + quant
        residual = residual - quant
        rr = jnp.sum(residual * residual, axis=-1, keepdims=True)
    out_ref[...] = acc
    i = pl.program_id(0)
    lv = jnp.reshape(loss, (1, 1))

    @pl.when(i == 0)
    def _init():
        loss_ref[...] = lv

    @pl.when(i > 0)
    def _accum():
        loss_ref[...] = loss_ref[...] + lv

    @pl.when(i == n_steps - 1)
    def _finish():
        loss_ref[...] = loss_ref[...] * inv_count


def kernel(faces, face_edges, codebooks):
    del face_edges
    b, n, d = faces.shape
    nq, k, _ = codebooks.shape
    m = b * n
    flat = faces.reshape(m, d)
    n_steps = m // _TILE
    c2, cbn, cbg = pl.pallas_call(
        _prep_kernel,
        out_shape=[
            jax.ShapeDtypeStruct((nq, 1, k), jnp.float32),
            jax.ShapeDtypeStruct((nq, k, d), jnp.float32),
            jax.ShapeDtypeStruct((nq, k, 2 * d), jnp.bfloat16),
        ],
    )(codebooks)
    out, loss = pl.pallas_call(
        functools.partial(
            _rvq_kernel, n_steps=n_steps, inv_count=1.0 / (m * d)
        ),
        grid=(n_steps,),
        in_specs=[
            pl.BlockSpec((_TILE, d), lambda i: (i, 0)),
            pl.BlockSpec((nq, 1, k), lambda i: (0, 0, 0)),
            pl.BlockSpec((nq, k, d), lambda i: (0, 0, 0)),
            pl.BlockSpec((nq, k, 2 * d), lambda i: (0, 0, 0)),
        ],
        out_specs=[
            pl.BlockSpec((_TILE, d), lambda i: (i, 0)),
            pl.BlockSpec((1, 1), lambda i: (0, 0)),
        ],
        out_shape=[
            jax.ShapeDtypeStruct((m, d), jnp.float32),
            jax.ShapeDtypeStruct((1, 1), jnp.float32),
        ],
        compiler_params=pltpu.CompilerParams(
            dimension_semantics=("arbitrary",),
        ),
    )(flat, c2, cbn, cbg)
    return out.reshape(b, n, d), loss[0, 0]
